# Initial kernel scaffold; baseline (speedup 1.0000x reference)
import jax
import jax.numpy as jnp
from jax.experimental import pallas as pl


def kernel(desc0, desc1, l0_W, l0_att_src, l0_att_dst, l0_bias, l0_c1W, l0_c1b, l0_bn_g, l0_bn_b, l0_bn_m, l0_bn_v, l0_c2W, l0_c2b, l1_W, l1_att_src, l1_att_dst, l1_bias, l1_c1W, l1_c1b, l1_bn_g, l1_bn_b, l1_bn_m, l1_bn_v, l1_c2W, l1_c2b):
    raise NotImplementedError("write your pallas kernel here")


# speedup vs baseline: 914.1350x; 914.1350x over previous
import jax
import jax.numpy as jnp
from jax.experimental import pallas as pl

_F = 256
_NG = 256
_B = 2
_N = 2 * _NG


def _lrelu(v):
    return jnp.where(v > 0, v, 0.2 * v)


def _layer(x, refs, cross):
    (W_ref, asrc_ref, adst_ref, bias_ref, c1W_ref, c1b_ref,
     bn_g_ref, bn_b_ref, bn_m_ref, bn_v_ref, c2W_ref, c2b_ref) = refs
    W = W_ref[...]
    h = jnp.dot(W, x, preferred_element_type=jnp.float32)
    hs = jnp.dot(asrc_ref[...], h, preferred_element_type=jnp.float32)
    hd = jnp.dot(adst_ref[...], h, preferred_element_type=jnp.float32)
    blocks = []
    for b in range(_B):
        for g in range(2):
            dcol = b * _N + g * _NG
            scol = b * _N + ((1 - g) * _NG if cross else g * _NG)
            hd_d = hd[:, dcol:dcol + _NG]
            hs_s = hs[:, scol:scol + _NG]
            h_s = h[:, scol:scol + _NG]
            logits = _lrelu(jnp.transpose(hd_d) + hs_s)
            if cross:
                hs_d = hs[:, dcol:dcol + _NG]
                lself = jnp.transpose(_lrelu(hs_d + hd_d))
                m = jnp.maximum(jnp.max(logits, axis=1, keepdims=True), lself)
                ex = jnp.exp(logits - m)
                exs = jnp.exp(lself - m)
                den = jnp.sum(ex, axis=1, keepdims=True) + exs + 1e-16
                num = jax.lax.dot_general(
                    h_s, ex, (((1,), (1,)), ((), ())),
                    preferred_element_type=jnp.float32)
                num = num + h[:, dcol:dcol + _NG] * jnp.transpose(exs)
                blocks.append(num / jnp.transpose(den))
            else:
                m = jnp.max(logits, axis=1, keepdims=True)
                ex = jnp.exp(logits - m)
                den = jnp.sum(ex, axis=1, keepdims=True) + 1e-16
                att = ex / den
                blocks.append(jax.lax.dot_general(
                    h_s, att, (((1,), (1,)), ((), ())),
                    preferred_element_type=jnp.float32))
    msg = jnp.concatenate(blocks, axis=1) + bias_ref[...]
    c1W = c1W_ref[...]
    y = (jnp.dot(c1W[:, :_F], x, preferred_element_type=jnp.float32)
         + jnp.dot(c1W[:, _F:], msg, preferred_element_type=jnp.float32)
         + c1b_ref[...])
    scale = bn_g_ref[...] * jax.lax.rsqrt(bn_v_ref[...] + 1e-5)
    y = (y - bn_m_ref[...]) * scale + bn_b_ref[...]
    y = jnp.maximum(y, 0.0)
    y2 = jnp.dot(c2W_ref[...], y, preferred_element_type=jnp.float32) + c2b_ref[...]
    return x + y2


def _fwd_kernel(*refs):
    x_ref = refs[0]
    l0 = refs[1:13]
    l1 = refs[13:25]
    out_ref = refs[25]
    x = x_ref[...]
    x = _layer(x, l0, cross=False)
    x = _layer(x, l1, cross=True)
    out_ref[...] = x


def kernel(desc0, desc1,
           l0_W, l0_att_src, l0_att_dst, l0_bias, l0_c1W, l0_c1b,
           l0_bn_g, l0_bn_b, l0_bn_m, l0_bn_v, l0_c2W, l0_c2b,
           l1_W, l1_att_src, l1_att_dst, l1_bias, l1_c1W, l1_c1b,
           l1_bn_g, l1_bn_b, l1_bn_m, l1_bn_v, l1_c2W, l1_c2b):
    n0 = desc0.shape[2]
    x = jnp.concatenate([desc0, desc1], axis=2)
    x = jnp.transpose(x, (1, 0, 2)).reshape(_F, _B * _N)

    def prep(W, asrc, adst, bias, c1W, c1b, g, b_, m, v, c2W, c2b):
        return (W, asrc.reshape(1, _F), adst.reshape(1, _F),
                bias.reshape(_F, 1), c1W, c1b.reshape(2 * _F, 1),
                g.reshape(2 * _F, 1), b_.reshape(2 * _F, 1),
                m.reshape(2 * _F, 1), v.reshape(2 * _F, 1),
                c2W, c2b.reshape(_F, 1))

    args = (x,
            *prep(l0_W, l0_att_src, l0_att_dst, l0_bias, l0_c1W, l0_c1b,
                  l0_bn_g, l0_bn_b, l0_bn_m, l0_bn_v, l0_c2W, l0_c2b),
            *prep(l1_W, l1_att_src, l1_att_dst, l1_bias, l1_c1W, l1_c1b,
                  l1_bn_g, l1_bn_b, l1_bn_m, l1_bn_v, l1_c2W, l1_c2b))

    out = pl.pallas_call(
        _fwd_kernel,
        out_shape=jax.ShapeDtypeStruct((_F, _B * _N), jnp.float32),
    )(*args)

    out = out.reshape(_F, _B, _N).transpose(1, 0, 2)
    return (out[:, :, :n0], out[:, :, n0:])


# speedup vs baseline: 961.4892x; 1.0518x over previous
import jax
import jax.numpy as jnp
from jax.experimental import pallas as pl
from jax.experimental.pallas import tpu as pltpu

_F = 256
_NG = 256
_B = 2
_N = 2 * _NG


def _lrelu(v):
    return jnp.where(v > 0, v, 0.2 * v)


def _layer(x, refs, cross):
    (W_ref, asrc_ref, adst_ref, bias_ref, c1W_ref, c1b_ref,
     bn_g_ref, bn_b_ref, bn_m_ref, bn_v_ref, c2W_ref, c2b_ref) = refs
    W = W_ref[...]
    h = jnp.dot(W, x, preferred_element_type=jnp.float32)
    hs = jnp.dot(asrc_ref[...], h, preferred_element_type=jnp.float32)
    hd = jnp.dot(adst_ref[...], h, preferred_element_type=jnp.float32)
    blocks = []
    for g in range(2):
        dcol = g * _NG
        scol = (1 - g) * _NG if cross else g * _NG
        hd_d = hd[:, dcol:dcol + _NG]
        hs_s = hs[:, scol:scol + _NG]
        h_s = h[:, scol:scol + _NG]
        logits = _lrelu(jnp.transpose(hd_d) + hs_s)
        if cross:
            hs_d = hs[:, dcol:dcol + _NG]
            lself = jnp.transpose(_lrelu(hs_d + hd_d))
            m = jnp.maximum(jnp.max(logits, axis=1, keepdims=True), lself)
            ex = jnp.exp(logits - m)
            exs = jnp.exp(lself - m)
            den = jnp.sum(ex, axis=1, keepdims=True) + exs + 1e-16
            num = jax.lax.dot_general(
                h_s, ex, (((1,), (1,)), ((), ())),
                preferred_element_type=jnp.float32)
            num = num + h[:, dcol:dcol + _NG] * jnp.transpose(exs)
            blocks.append(num / jnp.transpose(den))
        else:
            m = jnp.max(logits, axis=1, keepdims=True)
            ex = jnp.exp(logits - m)
            den = jnp.sum(ex, axis=1, keepdims=True) + 1e-16
            att = ex / den
            blocks.append(jax.lax.dot_general(
                h_s, att, (((1,), (1,)), ((), ())),
                preferred_element_type=jnp.float32))
    msg = jnp.concatenate(blocks, axis=1) + bias_ref[...]
    c1W = c1W_ref[...]
    y = (jnp.dot(c1W[:, :_F], x, preferred_element_type=jnp.float32)
         + jnp.dot(c1W[:, _F:], msg, preferred_element_type=jnp.float32)
         + c1b_ref[...])
    scale = bn_g_ref[...] * jax.lax.rsqrt(bn_v_ref[...] + 1e-5)
    y = (y - bn_m_ref[...]) * scale + bn_b_ref[...]
    y = jnp.maximum(y, 0.0)
    y2 = jnp.dot(c2W_ref[...], y, preferred_element_type=jnp.float32) + c2b_ref[...]
    return x + y2


def _fwd_kernel(*refs):
    d0_ref, d1_ref = refs[0], refs[1]
    l0 = refs[2:14]
    l1 = refs[14:26]
    out0_ref, out1_ref = refs[26], refs[27]
    x = jnp.concatenate([d0_ref[0], d1_ref[0]], axis=1)
    x = _layer(x, l0, cross=False)
    x = _layer(x, l1, cross=True)
    out0_ref[0] = x[:, :_NG]
    out1_ref[0] = x[:, _NG:]


def kernel(desc0, desc1,
           l0_W, l0_att_src, l0_att_dst, l0_bias, l0_c1W, l0_c1b,
           l0_bn_g, l0_bn_b, l0_bn_m, l0_bn_v, l0_c2W, l0_c2b,
           l1_W, l1_att_src, l1_att_dst, l1_bias, l1_c1W, l1_c1b,
           l1_bn_g, l1_bn_b, l1_bn_m, l1_bn_v, l1_c2W, l1_c2b):

    def prep(W, asrc, adst, bias, c1W, c1b, g, b_, m, v, c2W, c2b):
        return (W, asrc.reshape(1, _F), adst.reshape(1, _F),
                bias.reshape(_F, 1), c1W, c1b.reshape(2 * _F, 1),
                g.reshape(2 * _F, 1), b_.reshape(2 * _F, 1),
                m.reshape(2 * _F, 1), v.reshape(2 * _F, 1),
                c2W, c2b.reshape(_F, 1))

    params = (*prep(l0_W, l0_att_src, l0_att_dst, l0_bias, l0_c1W, l0_c1b,
                    l0_bn_g, l0_bn_b, l0_bn_m, l0_bn_v, l0_c2W, l0_c2b),
              *prep(l1_W, l1_att_src, l1_att_dst, l1_bias, l1_c1W, l1_c1b,
                    l1_bn_g, l1_bn_b, l1_bn_m, l1_bn_v, l1_c2W, l1_c2b))

    batch_spec = pl.BlockSpec((1, _F, _NG), lambda b: (b, 0, 0))
    const_specs = [pl.BlockSpec(p.shape, lambda b: (0,) * p.ndim)
                   for p in params]

    out0, out1 = pl.pallas_call(
        _fwd_kernel,
        grid=(_B,),
        in_specs=[batch_spec, batch_spec] + const_specs,
        out_specs=[batch_spec, batch_spec],
        out_shape=[jax.ShapeDtypeStruct((_B, _F, _NG), jnp.float32),
                   jax.ShapeDtypeStruct((_B, _F, _NG), jnp.float32)],
        compiler_params=pltpu.CompilerParams(
            dimension_semantics=("parallel",)),
    )(desc0, desc1, *params)
    return (out0, out1)


# speedup vs baseline: 993.8937x; 1.0337x over previous
import jax
import jax.numpy as jnp
from jax.experimental import pallas as pl
from jax.experimental.pallas import tpu as pltpu

_F = 256
_NG = 256
_B = 2
_N = 2 * _NG


def _lrelu(v):
    return jnp.where(v > 0, v, 0.2 * v)


def _layer(x, smalls, W, c1W, c2W, cross):
    (asrc_ref, adst_ref, bias_ref, c1b_ref,
     bn_g_ref, bn_b_ref, bn_m_ref, bn_v_ref, c2b_ref) = smalls
    h = jnp.dot(W, x, preferred_element_type=jnp.float32)
    hs = jnp.dot(asrc_ref[...], h, preferred_element_type=jnp.float32)
    hd = jnp.dot(adst_ref[...], h, preferred_element_type=jnp.float32)
    blocks = []
    for b in range(_B):
        for g in range(2):
            dcol = b * _N + g * _NG
            scol = b * _N + ((1 - g) * _NG if cross else g * _NG)
            hd_d = hd[:, dcol:dcol + _NG]
            hs_s = hs[:, scol:scol + _NG]
            h_s = h[:, scol:scol + _NG]
            logits = _lrelu(jnp.transpose(hd_d) + hs_s)
            if cross:
                hs_d = hs[:, dcol:dcol + _NG]
                lself = jnp.transpose(_lrelu(hs_d + hd_d))
                m = jnp.maximum(jnp.max(logits, axis=1, keepdims=True), lself)
                ex = jnp.exp(logits - m)
                exs = jnp.exp(lself - m)
                den = jnp.sum(ex, axis=1, keepdims=True) + exs + 1e-16
                r = 1.0 / den
                num = jax.lax.dot_general(
                    h_s, ex * r, (((1,), (1,)), ((), ())),
                    preferred_element_type=jnp.float32)
                blocks.append(num + h[:, dcol:dcol + _NG] * jnp.transpose(exs * r))
            else:
                m = jnp.max(logits, axis=1, keepdims=True)
                ex = jnp.exp(logits - m)
                den = jnp.sum(ex, axis=1, keepdims=True) + 1e-16
                att = ex / den
                blocks.append(jax.lax.dot_general(
                    h_s, att, (((1,), (1,)), ((), ())),
                    preferred_element_type=jnp.float32))
    msg = jnp.concatenate(blocks, axis=1) + bias_ref[...]
    y = (jnp.dot(c1W[:, :_F], x, preferred_element_type=jnp.float32)
         + jnp.dot(c1W[:, _F:], msg, preferred_element_type=jnp.float32)
         + c1b_ref[...])
    scale = bn_g_ref[...] * jax.lax.rsqrt(bn_v_ref[...] + 1e-5)
    y = (y - bn_m_ref[...]) * scale + bn_b_ref[...]
    y = jnp.maximum(y, 0.0)
    y2 = jnp.dot(c2W, y, preferred_element_type=jnp.float32) + c2b_ref[...]
    return x + y2


def _fwd_kernel(*refs):
    d0_ref, d1_ref = refs[0], refs[1]
    smalls0 = refs[2:11]
    smalls1 = refs[11:20]
    bigs = refs[20:26]
    out0_ref, out1_ref = refs[26], refs[27]
    vbufs = refs[28:34]
    sems = refs[34:40]
    copies = [pltpu.make_async_copy(bigs[i], vbufs[i], sems[i])
              for i in range(6)]
    for c in copies:
        c.start()
    x = jnp.concatenate([d0_ref[0], d1_ref[0], d0_ref[1], d1_ref[1]],
                        axis=1)
    for l, smalls in ((0, smalls0), (1, smalls1)):
        copies[3 * l].wait()
        W = vbufs[3 * l][...]
        copies[3 * l + 1].wait()
        c1W = vbufs[3 * l + 1][...]
        copies[3 * l + 2].wait()
        c2W = vbufs[3 * l + 2][...]
        x = _layer(x, smalls, W, c1W, c2W, cross=(l == 1))
    out0_ref[0] = x[:, 0 * _NG:1 * _NG]
    out1_ref[0] = x[:, 1 * _NG:2 * _NG]
    out0_ref[1] = x[:, 2 * _NG:3 * _NG]
    out1_ref[1] = x[:, 3 * _NG:4 * _NG]


def kernel(desc0, desc1,
           l0_W, l0_att_src, l0_att_dst, l0_bias, l0_c1W, l0_c1b,
           l0_bn_g, l0_bn_b, l0_bn_m, l0_bn_v, l0_c2W, l0_c2b,
           l1_W, l1_att_src, l1_att_dst, l1_bias, l1_c1W, l1_c1b,
           l1_bn_g, l1_bn_b, l1_bn_m, l1_bn_v, l1_c2W, l1_c2b):

    def smalls(asrc, adst, bias, c1b, g, b_, m, v, c2b):
        return (asrc.reshape(1, _F), adst.reshape(1, _F),
                bias.reshape(_F, 1), c1b.reshape(2 * _F, 1),
                g.reshape(2 * _F, 1), b_.reshape(2 * _F, 1),
                m.reshape(2 * _F, 1), v.reshape(2 * _F, 1),
                c2b.reshape(_F, 1))

    small_args = (*smalls(l0_att_src, l0_att_dst, l0_bias, l0_c1b,
                          l0_bn_g, l0_bn_b, l0_bn_m, l0_bn_v, l0_c2b),
                  *smalls(l1_att_src, l1_att_dst, l1_bias, l1_c1b,
                          l1_bn_g, l1_bn_b, l1_bn_m, l1_bn_v, l1_c2b))
    big_args = (l0_W, l0_c1W, l0_c2W, l1_W, l1_c1W, l1_c2W)

    vmem_spec = pl.BlockSpec(memory_space=pltpu.MemorySpace.VMEM)
    any_spec = pl.BlockSpec(memory_space=pltpu.MemorySpace.HBM)

    out0, out1 = pl.pallas_call(
        _fwd_kernel,
        in_specs=[vmem_spec] * (2 + len(small_args))
                 + [any_spec] * len(big_args),
        out_specs=[vmem_spec, vmem_spec],
        out_shape=[jax.ShapeDtypeStruct((_B, _F, _NG), jnp.float32),
                   jax.ShapeDtypeStruct((_B, _F, _NG), jnp.float32)],
        scratch_shapes=[pltpu.VMEM(b.shape, jnp.float32) for b in big_args]
                       + [pltpu.SemaphoreType.DMA] * len(big_args),
    )(desc0, desc1, *small_args, *big_args)
    return (out0, out1)


# speedup vs baseline: 2080.0594x; 2.0928x over previous
import jax
import jax.numpy as jnp
from jax.experimental import pallas as pl
from jax.experimental.pallas import tpu as pltpu

_F = 256
_NG = 256
_B = 2
_N = 2 * _NG


def _lrelu(v):
    return jnp.where(v > 0, v, 0.2 * v)


def _layer(x, smalls, W, c1W, c2W, cross):
    (asrc_ref, adst_ref, bias_ref, c1b_ref,
     bn_g_ref, bn_b_ref, bn_m_ref, bn_v_ref, c2b_ref) = smalls
    asrc = asrc_ref[...].reshape(1, _F)
    adst = adst_ref[...].reshape(1, _F)
    bias = bias_ref[...].reshape(_F, 1)
    c1b = c1b_ref[...].reshape(2 * _F, 1)
    bn_g = bn_g_ref[...].reshape(2 * _F, 1)
    bn_b = bn_b_ref[...].reshape(2 * _F, 1)
    bn_m = bn_m_ref[...].reshape(2 * _F, 1)
    bn_v = bn_v_ref[...].reshape(2 * _F, 1)
    c2b = c2b_ref[...].reshape(_F, 1)
    h = jnp.dot(W, x, preferred_element_type=jnp.float32)
    hs = jnp.dot(asrc, h, preferred_element_type=jnp.float32)
    hd = jnp.dot(adst, h, preferred_element_type=jnp.float32)
    blocks = []
    for b in range(_B):
        for g in range(2):
            dcol = b * _N + g * _NG
            scol = b * _N + ((1 - g) * _NG if cross else g * _NG)
            hd_d = hd[:, dcol:dcol + _NG]
            hs_s = hs[:, scol:scol + _NG]
            h_s = h[:, scol:scol + _NG]
            logits = _lrelu(jnp.transpose(hd_d) + hs_s)
            if cross:
                hs_d = hs[:, dcol:dcol + _NG]
                lself = jnp.transpose(_lrelu(hs_d + hd_d))
                m = jnp.maximum(jnp.max(logits, axis=1, keepdims=True), lself)
                ex = jnp.exp(logits - m)
                exs = jnp.exp(lself - m)
                den = jnp.sum(ex, axis=1, keepdims=True) + exs + 1e-16
                r = 1.0 / den
                num = jax.lax.dot_general(
                    h_s, ex * r, (((1,), (1,)), ((), ())),
                    preferred_element_type=jnp.float32)
                blocks.append(num + h[:, dcol:dcol + _NG] * jnp.transpose(exs * r))
            else:
                m = jnp.max(logits, axis=1, keepdims=True)
                ex = jnp.exp(logits - m)
                den = jnp.sum(ex, axis=1, keepdims=True) + 1e-16
                att = ex / den
                blocks.append(jax.lax.dot_general(
                    h_s, att, (((1,), (1,)), ((), ())),
                    preferred_element_type=jnp.float32))
    msg = jnp.concatenate(blocks, axis=1) + bias
    y = (jnp.dot(c1W[:, :_F], x, preferred_element_type=jnp.float32)
         + jnp.dot(c1W[:, _F:], msg, preferred_element_type=jnp.float32)
         + c1b)
    scale = bn_g * jax.lax.rsqrt(bn_v + 1e-5)
    y = (y - bn_m) * scale + bn_b
    y = jnp.maximum(y, 0.0)
    y2 = jnp.dot(c2W, y, preferred_element_type=jnp.float32) + c2b
    return x + y2


def _fwd_kernel(*refs):
    d0_ref, d1_ref = refs[0], refs[1]
    smalls0 = refs[2:11]
    smalls1 = refs[11:20]
    bigs = refs[20:26]
    out0_ref, out1_ref = refs[26], refs[27]
    vbufs = refs[28:34]
    sems = refs[34:40]
    copies = [pltpu.make_async_copy(bigs[i], vbufs[i], sems[i])
              for i in range(6)]
    for c in copies:
        c.start()
    x = jnp.concatenate([d0_ref[0], d1_ref[0], d0_ref[1], d1_ref[1]],
                        axis=1)
    for l, smalls in ((0, smalls0), (1, smalls1)):
        copies[3 * l].wait()
        W = vbufs[3 * l][...]
        copies[3 * l + 1].wait()
        c1W = vbufs[3 * l + 1][...]
        copies[3 * l + 2].wait()
        c2W = vbufs[3 * l + 2][...]
        x = _layer(x, smalls, W, c1W, c2W, cross=(l == 1))
    out0_ref[0] = x[:, 0 * _NG:1 * _NG]
    out1_ref[0] = x[:, 1 * _NG:2 * _NG]
    out0_ref[1] = x[:, 2 * _NG:3 * _NG]
    out1_ref[1] = x[:, 3 * _NG:4 * _NG]


def kernel(desc0, desc1,
           l0_W, l0_att_src, l0_att_dst, l0_bias, l0_c1W, l0_c1b,
           l0_bn_g, l0_bn_b, l0_bn_m, l0_bn_v, l0_c2W, l0_c2b,
           l1_W, l1_att_src, l1_att_dst, l1_bias, l1_c1W, l1_c1b,
           l1_bn_g, l1_bn_b, l1_bn_m, l1_bn_v, l1_c2W, l1_c2b):

    small_args = (l0_att_src, l0_att_dst, l0_bias, l0_c1b,
                  l0_bn_g, l0_bn_b, l0_bn_m, l0_bn_v, l0_c2b,
                  l1_att_src, l1_att_dst, l1_bias, l1_c1b,
                  l1_bn_g, l1_bn_b, l1_bn_m, l1_bn_v, l1_c2b)
    big_args = (l0_W, l0_c1W, l0_c2W, l1_W, l1_c1W, l1_c2W)

    vmem_spec = pl.BlockSpec(memory_space=pltpu.MemorySpace.VMEM)
    any_spec = pl.BlockSpec(memory_space=pltpu.MemorySpace.HBM)

    out0, out1 = pl.pallas_call(
        _fwd_kernel,
        in_specs=[vmem_spec] * (2 + len(small_args))
                 + [any_spec] * len(big_args),
        out_specs=[vmem_spec, vmem_spec],
        out_shape=[jax.ShapeDtypeStruct((_B, _F, _NG), jnp.float32),
                   jax.ShapeDtypeStruct((_B, _F, _NG), jnp.float32)],
        scratch_shapes=[pltpu.VMEM(b.shape, jnp.float32) for b in big_args]
                       + [pltpu.SemaphoreType.DMA] * len(big_args),
    )(desc0, desc1, *small_args, *big_args)
    return (out0, out1)
